# Initial kernel scaffold; baseline (speedup 1.0000x reference)
import jax
import jax.numpy as jnp
from jax.experimental import pallas as pl


def kernel(input_ids, table):
    raise NotImplementedError("write your pallas kernel here")


# speedup vs baseline: 1.5461x; 1.5461x over previous
import functools

import jax
from jax import lax
import jax.numpy as jnp
from jax.experimental import pallas as pl
from jax.experimental.pallas import tpu as pltpu
from jax.experimental.pallas import tpu_sc as plsc

NC, NS = 2, 16
NW = NC * NS
CHUNK = 64


def kernel(input_ids, table):
    batch, seq = input_ids.shape
    n = batch * seq
    _, d = table.shape
    b_per_w = n // NW
    nchunk = b_per_w // CHUNK
    idx = input_ids.reshape(n).astype(jnp.int32)

    mesh = plsc.VectorSubcoreMesh(core_axis_name="c", subcore_axis_name="s")

    @functools.partial(
        pl.kernel,
        out_type=jax.ShapeDtypeStruct((n, d), table.dtype),
        mesh=mesh,
        scratch_types=[
            pltpu.VMEM((b_per_w,), jnp.int32),
            pltpu.VMEM((2, CHUNK, d), jnp.float32),
            pltpu.SemaphoreType.DMA,
        ],
    )
    def gather_kernel(tab_hbm, idx_hbm, out_hbm, idx_v, rows_v, sem):
        wid = lax.axis_index("s") * NC + lax.axis_index("c")
        base = wid * b_per_w
        pltpu.sync_copy(idx_hbm.at[pl.ds(base, b_per_w)], idx_v)

        @pl.loop(0, nchunk)
        def _(ci):
            off = ci * CHUNK
            pltpu.async_copy(tab_hbm.at[idx_v.at[pl.ds(off, CHUNK)]],
                             rows_v.at[0], sem).wait()
            pltpu.sync_copy(rows_v.at[0], out_hbm.at[pl.ds(base + off, CHUNK)])

    out = gather_kernel(table, idx)
    return out.reshape(batch, seq, d)


# speedup vs baseline: 1.7025x; 1.1012x over previous
import functools

import jax
from jax import lax
import jax.numpy as jnp
from jax.experimental import pallas as pl
from jax.experimental.pallas import tpu as pltpu
from jax.experimental.pallas import tpu_sc as plsc

NC, NS = 2, 16
NW = NC * NS
CHUNK = 64


def kernel(input_ids, table):
    batch, seq = input_ids.shape
    n = batch * seq
    _, d = table.shape
    b_per_w = n // NW
    nchunk = b_per_w // CHUNK
    idx = input_ids.reshape(n).astype(jnp.int32)

    mesh = plsc.VectorSubcoreMesh(core_axis_name="c", subcore_axis_name="s")

    @functools.partial(
        pl.kernel,
        out_type=jax.ShapeDtypeStruct((n, d), table.dtype),
        mesh=mesh,
        scratch_types=[
            pltpu.VMEM((b_per_w,), jnp.int32),
            pltpu.VMEM((2, CHUNK, d), jnp.float32),
            pltpu.SemaphoreType.DMA,
            pltpu.SemaphoreType.DMA,
            pltpu.SemaphoreType.DMA,
            pltpu.SemaphoreType.DMA,
        ],
    )
    def gather_kernel(tab_hbm, idx_hbm, out_hbm, idx_v, rows_v, g0, g1, o0, o1):
        gsems = (g0, g1)
        osems = (o0, o1)
        wid = lax.axis_index("s") * NC + lax.axis_index("c")
        base = wid * b_per_w
        pltpu.sync_copy(idx_hbm.at[pl.ds(base, b_per_w)], idx_v)

        def gather_cp(g, b):
            return pltpu.make_async_copy(
                tab_hbm.at[idx_v.at[pl.ds(g * CHUNK, CHUNK)]],
                rows_v.at[b], gsems[b])

        def out_cp(g, b):
            return pltpu.make_async_copy(
                rows_v.at[b], out_hbm.at[pl.ds(base + g * CHUNK, CHUNK)],
                osems[b])

        gather_cp(0, 0).start()
        gather_cp(1, 1).start()

        @pl.loop(0, nchunk, step=2)
        def _(c):
            for b in range(2):
                g = c + b
                gather_cp(g, b).wait()
                out_cp(g, b).start()
                out_cp(g, b).wait()

                @pl.when(g + 2 < nchunk)
                def _():
                    gather_cp(g + 2, b).start()

    out = gather_kernel(table, idx)
    return out.reshape(batch, seq, d)


# speedup vs baseline: 1.7226x; 1.0119x over previous
import functools

import jax
from jax import lax
import jax.numpy as jnp
from jax.experimental import pallas as pl
from jax.experimental.pallas import tpu as pltpu
from jax.experimental.pallas import tpu_sc as plsc

NC, NS = 2, 16
NW = NC * NS
CHUNK = 32
NBUF = 4


def kernel(input_ids, table):
    batch, seq = input_ids.shape
    n = batch * seq
    _, d = table.shape
    b_per_w = n // NW
    nchunk = b_per_w // CHUNK
    idx = input_ids.reshape(n).astype(jnp.int32)

    mesh = plsc.VectorSubcoreMesh(core_axis_name="c", subcore_axis_name="s")

    @functools.partial(
        pl.kernel,
        out_type=jax.ShapeDtypeStruct((n, d), table.dtype),
        mesh=mesh,
        scratch_types=[
            pltpu.VMEM((b_per_w,), jnp.int32),
            pltpu.VMEM((NBUF, CHUNK, d), jnp.float32),
        ] + [pltpu.SemaphoreType.DMA] * (2 * NBUF),
    )
    def gather_kernel(tab_hbm, idx_hbm, out_hbm, idx_v, rows_v, *sems):
        gsems = sems[:NBUF]
        osems = sems[NBUF:]
        wid = lax.axis_index("s") * NC + lax.axis_index("c")
        base = wid * b_per_w
        pltpu.sync_copy(idx_hbm.at[pl.ds(base, b_per_w)], idx_v)

        def gather_cp(g, b):
            return pltpu.make_async_copy(
                tab_hbm.at[idx_v.at[pl.ds(g * CHUNK, CHUNK)]],
                rows_v.at[b], gsems[b])

        def out_cp(g, b):
            return pltpu.make_async_copy(
                rows_v.at[b], out_hbm.at[pl.ds(base + g * CHUNK, CHUNK)],
                osems[b])

        for b in range(NBUF):
            gather_cp(b, b).start()

        @pl.loop(0, nchunk, step=NBUF)
        def _(c):
            for b in range(NBUF):
                g = c + b
                gather_cp(g, b).wait()
                out_cp(g, b).start()

                @pl.when(g + NBUF < nchunk)
                def _():
                    out_cp(g, b).wait()
                    gather_cp(g + NBUF, b).start()

        for b in range(NBUF):
            out_cp(nchunk - NBUF + b, b).wait()

    out = gather_kernel(table, idx)
    return out.reshape(batch, seq, d)
